# baseline (speedup 1.0000x reference)
import functools

import jax
import jax.numpy as jnp
from jax import lax
from jax.experimental import pallas as pl
from jax.experimental.pallas import tpu as pltpu
from jax.experimental.pallas import tpu_sc as plsc

N_IN = 163842
D = 512
NUM_NODES = (N_IN + 6) // 4

NB = 8
NBUF = 2
RSPLIT = 1
ROWL = D // RSPLIT
RPB = NB * 7 * RSPLIT
SPLITS = (RPB,)
NC = 2
NS = 16
NW = NC * NS
NBW = 162
NBLK = NW * NBW
NPAD = NBLK * NB
IDX_W = NBW * RPB
IDX_FETCH = (NBW + NBUF) * RPB
IDX_PAD = (NW - 1) * IDX_W + IDX_FETCH


@functools.partial(
    pl.kernel,
    mesh=plsc.VectorSubcoreMesh(core_axis_name="c", subcore_axis_name="s"),
    compiler_params=pltpu.CompilerParams(needs_layout_passes=False),
    out_type=jax.ShapeDtypeStruct((NPAD * D,), jnp.float32),
    scratch_types=[
        pltpu.VMEM((IDX_FETCH,), jnp.int32),
        pltpu.VMEM((RPB, ROWL), jnp.float32),
        pltpu.VMEM((RPB, ROWL), jnp.float32),
        pltpu.VMEM((RPB, ROWL), jnp.float32),
        pltpu.VMEM((NB * D,), jnp.float32),
        pltpu.VMEM((NB * D,), jnp.float32),
        pltpu.VMEM((NB * D,), jnp.float32),
        pltpu.SemaphoreType.DMA,
        pltpu.SemaphoreType.DMA,
        pltpu.SemaphoreType.DMA,
        pltpu.SemaphoreType.DMA,
        pltpu.SemaphoreType.DMA,
        pltpu.SemaphoreType.DMA,
    ],
)
def _pool_sc(x_hbm, idx_hbm, out_hbm, idx_v, rows0, rows1, rows2,
             o0, o1, o2, rs0, rs1, rs2, os0, os1, os2):
    rows = [rows0, rows1, rows2][:NBUF]
    outs = [o0, o1, o2][:NBUF]
    rsems = [rs0, rs1, rs2][:NBUF]
    osems = [os0, os1, os2][:NBUF]
    wid = lax.axis_index("s") * NC + lax.axis_index("c")
    iota7 = lax.iota(jnp.int32, 16) * 7
    bases = [iota7 + n * (7 * D) for n in range(NB)]

    pltpu.sync_copy(idx_hbm.at[pl.ds(wid * IDX_W, IDX_FETCH)], idx_v)

    def gather_rows(blk, buf, sem):
        off = 0
        for sz in SPLITS:
            pltpu.async_copy(
                x_hbm.at[idx_v.at[pl.ds(blk * RPB + off, sz)]],
                buf.at[pl.ds(off, sz)], sem)
            off += sz

    def wait_rows(blk, buf, sem):
        pltpu.make_async_copy(
            x_hbm.at[idx_v.at[pl.ds(blk * RPB, RPB)]], buf, sem
        ).wait()

    def pool_block(buf, out_v):
        @plsc.parallel_loop(0, 32, unroll=3)
        def chunk(j):
            off = j * 112
            for n in range(NB):
                pv = bases[n] + off
                acc = plsc.load_gather(buf, [pv >> 9, pv & 511])
                for k in range(1, 7):
                    pk = pv + k
                    acc = acc + plsc.load_gather(buf, [pk >> 9, pk & 511])
                out_v[pl.ds(n * D + j * 16, 16)] = acc * (1.0 / 7.0)

    def out_ref(blk):
        g = wid * NBW + blk
        return out_hbm.at[pl.ds(g * (NB * D), NB * D)]

    def copy_out(blk, out_v, sem):
        pltpu.async_copy(out_v, out_ref(blk), sem)

    def wait_out(blk, out_v, sem):
        pltpu.make_async_copy(out_v, out_ref(blk), sem).wait()

    for s in range(NBUF):
        gather_rows(s, rows[s], rsems[s])

    def step(t, _):
        b0 = NBUF * t
        for s in range(NBUF):
            wait_rows(b0 + s, rows[s], rsems[s])
            pool_block(rows[s], outs[s])
            gather_rows(b0 + s + NBUF, rows[s], rsems[s])
            copy_out(b0 + s, outs[s], osems[s])
        for s in range(NBUF):
            wait_out(b0 + s, outs[s], osems[s])
        return 0

    lax.fori_loop(0, NBW // NBUF, step, 0)


def kernel(x, neigh_orders):
    idx = neigh_orders[: NUM_NODES * 7]
    if RSPLIT > 1:
        idx = (idx[:, None] * RSPLIT
               + jnp.arange(RSPLIT, dtype=jnp.int32)[None, :]).reshape(-1)
    x2 = x.reshape(N_IN * RSPLIT, ROWL)
    idx = jnp.concatenate(
        [idx, jnp.zeros((IDX_PAD - idx.shape[0],), jnp.int32)])
    out_flat = _pool_sc(x2, idx)
    return out_flat.reshape(NPAD, D)[:NUM_NODES]


# speedup vs baseline: 1.0446x; 1.0446x over previous
import functools

import jax
import jax.numpy as jnp
from jax import lax
from jax.experimental import pallas as pl
from jax.experimental.pallas import tpu as pltpu
from jax.experimental.pallas import tpu_sc as plsc

N_IN = 163842
D = 512
NUM_NODES = (N_IN + 6) // 4

NB = 8
NBUF = 2
RPB = NB * 7
NC = 2
NS = 16
NW = NC * NS
NBW = 162
NBLK = NW * NBW
NPAD = NBLK * NB
IDX_W = NBW * RPB
IDX_FETCH = (NBW + NBUF) * RPB
IDX_PAD = (NW - 1) * IDX_W + IDX_FETCH


@functools.partial(
    pl.kernel,
    mesh=plsc.VectorSubcoreMesh(core_axis_name="c", subcore_axis_name="s"),
    compiler_params=pltpu.CompilerParams(needs_layout_passes=False),
    out_type=jax.ShapeDtypeStruct((NPAD * D,), jnp.float32),
    scratch_types=[
        pltpu.VMEM((IDX_FETCH,), jnp.int32),
        pltpu.VMEM((RPB, D), jnp.float32),
        pltpu.VMEM((RPB, D), jnp.float32),
        pltpu.VMEM((NB * D,), jnp.float32),
        pltpu.VMEM((NB * D,), jnp.float32),
        pltpu.SemaphoreType.DMA,
        pltpu.SemaphoreType.DMA,
        pltpu.SemaphoreType.DMA,
        pltpu.SemaphoreType.DMA,
    ],
)
def _pool_sc(x_hbm, idx_hbm, out_hbm, idx_v, rows0, rows1,
             o0, o1, rs0, rs1, os0, os1):
    rows = [rows0, rows1]
    outs = [o0, o1]
    rsems = [rs0, rs1]
    osems = [os0, os1]
    wid = lax.axis_index("s") * NC + lax.axis_index("c")
    iota7 = lax.iota(jnp.int32, 16) * 7
    bases = [iota7 + n * (7 * D) for n in range(NB)]

    pltpu.sync_copy(idx_hbm.at[pl.ds(wid * IDX_W, IDX_FETCH)], idx_v)

    def gather_rows(blk, buf, sem):
        pltpu.async_copy(
            x_hbm.at[idx_v.at[pl.ds(blk * RPB, RPB)]], buf, sem)

    def wait_rows(blk, buf, sem):
        pltpu.make_async_copy(
            x_hbm.at[idx_v.at[pl.ds(blk * RPB, RPB)]], buf, sem
        ).wait()

    def pool_block(buf, out_v):
        @plsc.parallel_loop(0, 32, unroll=2)
        def chunk(j):
            off = j * 112
            for n in range(NB):
                pv = bases[n] + off
                acc = plsc.load_gather(buf, [pv >> 9, pv & 511])
                for k in range(1, 7):
                    pk = pv + k
                    acc = acc + plsc.load_gather(buf, [pk >> 9, pk & 511])
                out_v[pl.ds(n * D + j * 16, 16)] = acc * (1.0 / 7.0)

    def out_ref(blk):
        g = wid * NBW + blk
        return out_hbm.at[pl.ds(g * (NB * D), NB * D)]

    def copy_out(blk, out_v, sem):
        pltpu.async_copy(out_v, out_ref(blk), sem)

    def wait_out(blk, out_v, sem):
        pltpu.make_async_copy(out_v, out_ref(blk), sem).wait()

    for s in range(NBUF):
        gather_rows(s, rows[s], rsems[s])

    def step(t, _):
        b0 = NBUF * t
        for s in range(NBUF):
            wait_rows(b0 + s, rows[s], rsems[s])
            pool_block(rows[s], outs[s])
            gather_rows(b0 + s + NBUF, rows[s], rsems[s])
            copy_out(b0 + s, outs[s], osems[s])
        for s in range(NBUF):
            wait_out(b0 + s, outs[s], osems[s])
        return 0

    lax.fori_loop(0, NBW // NBUF, step, 0)


def kernel(x, neigh_orders):
    idx = neigh_orders[: NUM_NODES * 7]
    idx = jnp.concatenate(
        [idx, jnp.zeros((IDX_PAD - NUM_NODES * 7,), jnp.int32)])
    out_flat = _pool_sc(x, idx)
    return out_flat.reshape(NPAD, D)[:NUM_NODES]
